# baseline (speedup 1.0000x reference)
import functools
import math

import jax
import jax.numpy as jnp
from jax import lax
from jax.experimental import pallas as pl
from jax.experimental.pallas import tpu as pltpu
from jax.experimental.pallas import tpu_sc as plsc

N, T, D = 4096, 16, 128
KEY_DIM = 64
TOPK = 8
BLK = 256
SCALE = 1.0 / math.sqrt(KEY_DIM)

CHUNK = 128
NCHUNK = N // CHUNK
HN = N // 2
HN_LOG = 11
HBLK = HN // BLK
NW = 32
RW = HN // NW
BATCH = 32
NBATCH = RW // BATCH
L = 16
NEG = -3.0e38


def _pool_proj_body(u_ref, wq_ref, bq_ref, wk_ref, bk_ref, q_ref, kt_ref):
    pool = jnp.mean(u_ref[...], axis=1)
    cdims = (((1,), (1,)), ((), ()))
    q_ref[...] = lax.dot_general(pool, wq_ref[...], cdims,
                                 preferred_element_type=jnp.float32) + bq_ref[...]
    k = lax.dot_general(pool, wk_ref[...], cdims,
                        preferred_element_type=jnp.float32) + bk_ref[...]
    kt_ref[...] = k.T


def _scores_body(q_ref, kt_ref, s_ref, cm_ref):
    cms = []
    for c in range(NCHUNK):
        kc = kt_ref[:, c * CHUNK:(c + 1) * CHUNK]
        sc = jnp.dot(q_ref[...], kc,
                     preferred_element_type=jnp.float32) * SCALE
        s_ref[c, :, :] = sc
        cms.append(jnp.max(sc, axis=1, keepdims=True))
    cm_ref[...] = jnp.concatenate(cms, axis=1)


def _hi8(vec):
    return lax.rev(vec, (0,))


def _sc_select_body(cm_hbm, s_hbm, cols_hbm, vals_hbm,
                    cm_v, idx_v, t_v, cand_v, oc_v, ov_v, sem):
    nc = 2
    wid = lax.axis_index("s") * nc + lax.axis_index("c")
    base = wid * RW

    pltpu.sync_copy(cm_hbm.at[pl.ds(base * NCHUNK, RW * NCHUNK)], cm_v)

    lane = lax.iota(jnp.int32, L)
    lane_lt8 = lane < TOPK

    def phase_a(rl, rbase):
        r = rbase + rl
        cm0 = cm_v[pl.ds(r * NCHUNK, L)]
        cm1 = cm_v[pl.ds(r * NCHUNK + L, L)]
        s0, i0 = plsc.sort_key_val(cm0, lane, descending=True)
        s1, i1 = plsc.sort_key_val(cm1, lane + L, descending=True)
        mv = jnp.where(lane_lt8, s0, _hi8(s1))
        mi = jnp.where(lane_lt8, i0, _hi8(i1))
        sv, si = plsc.sort_key_val(mv, mi, descending=True)
        t = jnp.max(jnp.where(lane == TOPK - 1, sv, NEG))
        t_v[pl.ds(rl * L, L)] = jnp.full((L,), t, jnp.float32)
        gidx = si * HN + (base + r)
        plsc.store_compressed(idx_v.at[pl.ds(rl * TOPK, L)], gidx, mask=lane_lt8)
        return rbase

    def _merge_tree(tops):
        while len(tops) > 1:
            nxt = []
            for a, b in zip(tops[0::2], tops[1::2]):
                cv = jnp.where(lane_lt8, a[0], _hi8(b[0]))
                ci = jnp.where(lane_lt8, a[1], _hi8(b[1]))
                res = plsc.sort_key_val(cv, ci, descending=True)
                nxt.append((res[0], res[1]))
            tops = nxt
        return tops[0]

    def phase_b(rl, rbase):
        r = rbase + rl
        t = t_v[pl.ds(rl * L, L)]
        gv = idx_v[pl.ds(rl * TOPK, L)]
        colbases = []
        chunk_tops = []
        bad = jnp.zeros((L,), jnp.int32)
        for j in range(TOPK):
            sid = gv[j]
            colbase = ((sid - (base + r)) >> HN_LOG) * CHUNK
            colbases.append(colbase)
            vs = [cand_v[rl * TOPK + j, pl.ds(qq * L, L)]
                  for qq in range(CHUNK // L)]
            m = vs[0]
            mq = jnp.zeros((L,), jnp.int32)
            cnt = (vs[0] >= t).astype(jnp.int32)
            for qq in range(1, CHUNK // L):
                gt = vs[qq] > m
                m = jnp.where(gt, vs[qq], m)
                mq = jnp.where(gt, qq, mq)
                cnt = cnt + (vs[qq] >= t).astype(jnp.int32)
            bad = jnp.maximum(bad, cnt)
            vm = jnp.where(m >= t, m, NEG)
            im = colbase + mq * L + lane
            res = plsc.sort_key_val(vm, im, descending=True)
            chunk_tops.append((res[0], res[1]))
        fast = _merge_tree(chunk_tops)

        def full_row_top(args, t=t, colbases=colbases, rl=rl):
            del args
            tops = []
            for j in range(TOPK):
                av = jnp.full((L,), NEG, jnp.float32)
                ai = jnp.zeros((L,), jnp.int32)
                for qq in range(CHUNK // L):
                    v = cand_v[rl * TOPK + j, pl.ds(qq * L, L)]
                    vm = jnp.where(v >= t, v, NEG)
                    im = colbases[j] + qq * L + lane
                    sv2, si2 = plsc.sort_key_val(vm, im, descending=True)
                    cv = jnp.where(lane_lt8, av, _hi8(sv2))
                    ci = jnp.where(lane_lt8, ai, _hi8(si2))
                    av, ai = plsc.sort_key_val(cv, ci, descending=True)
                tops.append((av, ai))
            return _merge_tree(tops)

        multi = plsc.all_reduce_population_count(bad >= 2)[0] > 0
        bv, bi = lax.cond(multi, full_row_top, lambda args: args, fast)
        e = jnp.where(lane_lt8, jnp.exp(bv - jnp.max(bv)), 0.0)
        p = e / jnp.sum(e)
        key = jnp.where(lane_lt8, bi, jnp.int32(N))
        sk, sp = plsc.sort_key_val(key, p, descending=False)
        plsc.store_compressed(oc_v.at[pl.ds(r * TOPK, L)], sk, mask=lane_lt8)
        plsc.store_compressed(ov_v.at[pl.ds(r * TOPK, L)], sp, mask=lane_lt8)
        return rbase

    def phase_a2(i, rb):
        phase_a(2 * i, rb)
        return phase_a(2 * i + 1, rb)

    def phase_b2(i, rb):
        phase_b(2 * i, rb)
        return phase_b(2 * i + 1, rb)

    half = BATCH * TOPK // 2
    for b in range(NBATCH):
        rbase = b * BATCH
        lax.fori_loop(0, BATCH // 2, phase_a2, rbase)
        c1 = pltpu.async_copy(s_hbm.at[idx_v.at[pl.ds(0, half)]],
                              cand_v.at[pl.ds(0, half)], sem)
        c2 = pltpu.async_copy(s_hbm.at[idx_v.at[pl.ds(half, half)]],
                              cand_v.at[pl.ds(half, half)], sem)
        c1.wait()
        c2.wait()
        lax.fori_loop(0, BATCH // 2, phase_b2, rbase)

    pltpu.sync_copy(oc_v.at[pl.ds(0, RW * TOPK)],
                    cols_hbm.at[pl.ds(base * TOPK, RW * TOPK)])
    pltpu.sync_copy(ov_v.at[pl.ds(0, RW * TOPK)],
                    vals_hbm.at[pl.ds(base * TOPK, RW * TOPK)])


_sc_select = functools.partial(
    pl.kernel,
    out_type=[jax.ShapeDtypeStruct((HN * TOPK,), jnp.int32),
              jax.ShapeDtypeStruct((HN * TOPK,), jnp.float32)],
    mesh=plsc.VectorSubcoreMesh(core_axis_name="c", subcore_axis_name="s"),
    compiler_params=pltpu.CompilerParams(needs_layout_passes=False),
    scratch_types=[
        pltpu.VMEM((RW * NCHUNK,), jnp.float32),
        pltpu.VMEM((BATCH * TOPK + L,), jnp.int32),
        pltpu.VMEM((BATCH * L,), jnp.float32),
        pltpu.VMEM((BATCH * TOPK, CHUNK), jnp.float32),
        pltpu.VMEM((RW * TOPK + L,), jnp.int32),
        pltpu.VMEM((RW * TOPK + L,), jnp.float32),
        pltpu.SemaphoreType.DMA,
    ],
)(_sc_select_body)


def _scores_half(q_half, kt):
    return pl.pallas_call(
        _scores_body,
        grid=(HBLK,),
        in_specs=[
            pl.BlockSpec((BLK, KEY_DIM), lambda i: (i, 0)),
            pl.BlockSpec((KEY_DIM, N), lambda i: (0, 0)),
        ],
        out_specs=[
            pl.BlockSpec((NCHUNK, BLK, CHUNK), lambda i: (0, i, 0)),
            pl.BlockSpec((BLK, NCHUNK), lambda i: (i, 0)),
        ],
        out_shape=[
            jax.ShapeDtypeStruct((NCHUNK, HN, CHUNK), jnp.float32),
            jax.ShapeDtypeStruct((HN, NCHUNK), jnp.float32),
        ],
    )(q_half, kt)


@jax.jit
def kernel(U, Wq, bq, Wk, bk):
    q, kt = pl.pallas_call(
        _pool_proj_body,
        grid=(N // BLK,),
        in_specs=[
            pl.BlockSpec((BLK, T, D), lambda i: (i, 0, 0)),
            pl.BlockSpec((KEY_DIM, D), lambda i: (0, 0)),
            pl.BlockSpec((1, KEY_DIM), lambda i: (0, 0)),
            pl.BlockSpec((KEY_DIM, D), lambda i: (0, 0)),
            pl.BlockSpec((1, KEY_DIM), lambda i: (0, 0)),
        ],
        out_specs=[
            pl.BlockSpec((BLK, KEY_DIM), lambda i: (i, 0)),
            pl.BlockSpec((KEY_DIM, BLK), lambda i: (0, i)),
        ],
        out_shape=[
            jax.ShapeDtypeStruct((N, KEY_DIM), jnp.float32),
            jax.ShapeDtypeStruct((KEY_DIM, N), jnp.float32),
        ],
    )(U, Wq, bq.reshape(1, KEY_DIM), Wk, bk.reshape(1, KEY_DIM))

    cols_parts = []
    vals_parts = []
    for h in range(N // HN):
        sh, cmh = _scores_half(q[h * HN:(h + 1) * HN], kt)
        ch, vh = _sc_select(cmh.reshape(-1), sh.reshape(NCHUNK * HN, CHUNK))
        cols_parts.append(ch)
        vals_parts.append(vh)
    cols = jnp.concatenate(cols_parts)
    vals = jnp.concatenate(vals_parts)
    rows = jnp.repeat(jnp.arange(N, dtype=jnp.int32), TOPK)
    indices = jnp.stack([rows.astype(jnp.int64),
                         cols.astype(jnp.int64)], axis=0)
    return indices, vals


# speedup vs baseline: 1.0579x; 1.0579x over previous
import functools
import math

import jax
import jax.numpy as jnp
from jax import lax
from jax.experimental import pallas as pl
from jax.experimental.pallas import tpu as pltpu
from jax.experimental.pallas import tpu_sc as plsc

N, T, D = 4096, 16, 128
KEY_DIM = 64
TOPK = 8
BLK = 256
SCALE = 1.0 / math.sqrt(KEY_DIM)

CHUNK = 128
NCHUNK = N // CHUNK
HN = N // 2
HN_LOG = 11
HBLK = HN // BLK
NW = 32
RW = HN // NW
BATCH = 32
NBATCH = RW // BATCH
L = 16
NEG = -3.0e38


def _pool_proj_body(u_ref, wq_ref, bq_ref, wk_ref, bk_ref, q_ref, kt_ref):
    pool = jnp.mean(u_ref[...], axis=1)
    cdims = (((1,), (1,)), ((), ()))
    q_ref[...] = lax.dot_general(pool, wq_ref[...], cdims,
                                 preferred_element_type=jnp.float32) + bq_ref[...]
    k = lax.dot_general(pool, wk_ref[...], cdims,
                        preferred_element_type=jnp.float32) + bk_ref[...]
    kt_ref[...] = k.T


def _scores_body(q_ref, kt_ref, s_ref, cm_ref):
    cms = []
    for c in range(NCHUNK):
        kc = kt_ref[:, c * CHUNK:(c + 1) * CHUNK]
        sc = jnp.dot(q_ref[...], kc,
                     preferred_element_type=jnp.float32) * SCALE
        s_ref[c, :, :] = sc
        cms.append(jnp.max(sc, axis=1, keepdims=True))
    cm_ref[...] = jnp.concatenate(cms, axis=1)


def _hi8(vec):
    return lax.rev(vec, (0,))


def _sc_select_body(cm_hbm, s_hbm, cols_hbm, vals_hbm,
                    cm_v, idx_v, t_v, cand_v, oc_v, ov_v, sem):
    nc = 2
    wid = lax.axis_index("s") * nc + lax.axis_index("c")
    base = wid * RW

    pltpu.sync_copy(cm_hbm.at[pl.ds(base * NCHUNK, RW * NCHUNK)], cm_v)

    lane = lax.iota(jnp.int32, L)
    lane_lt8 = lane < TOPK

    def phase_a(rl, rbase):
        r = rbase + rl
        cm0 = cm_v[pl.ds(r * NCHUNK, L)]
        cm1 = cm_v[pl.ds(r * NCHUNK + L, L)]
        s0, i0 = plsc.sort_key_val(cm0, lane, descending=True)
        s1, i1 = plsc.sort_key_val(cm1, lane + L, descending=True)
        mv = jnp.where(lane_lt8, s0, _hi8(s1))
        mi = jnp.where(lane_lt8, i0, _hi8(i1))
        sv, si = plsc.sort_key_val(mv, mi, descending=True)
        t = jnp.max(jnp.where(lane == TOPK - 1, sv, NEG))
        t_v[pl.ds(rl * L, L)] = jnp.full((L,), t, jnp.float32)
        gidx = si * HN + (base + r)
        plsc.store_compressed(idx_v.at[pl.ds(rl * TOPK, L)], gidx, mask=lane_lt8)
        return rbase

    def _merge_tree(tops):
        while len(tops) > 1:
            nxt = []
            for a, b in zip(tops[0::2], tops[1::2]):
                cv = jnp.where(lane_lt8, a[0], _hi8(b[0]))
                ci = jnp.where(lane_lt8, a[1], _hi8(b[1]))
                res = plsc.sort_key_val(cv, ci, descending=True)
                nxt.append((res[0], res[1]))
            tops = nxt
        return tops[0]

    def phase_b(rl, rbase):
        r = rbase + rl
        t = t_v[pl.ds(rl * L, L)]
        gv = idx_v[pl.ds(rl * TOPK, L)]
        colbases = []
        chunk_tops = []
        bad = jnp.zeros((L,), jnp.int32)
        for j in range(TOPK):
            sid = gv[j]
            colbase = ((sid - (base + r)) >> HN_LOG) * CHUNK
            colbases.append(colbase)
            vs = [cand_v[rl * TOPK + j, pl.ds(qq * L, L)]
                  for qq in range(CHUNK // L)]
            m = vs[0]
            mq = jnp.zeros((L,), jnp.int32)
            cnt = (vs[0] >= t).astype(jnp.int32)
            for qq in range(1, CHUNK // L):
                gt = vs[qq] > m
                m = jnp.where(gt, vs[qq], m)
                mq = jnp.where(gt, qq, mq)
                cnt = cnt + (vs[qq] >= t).astype(jnp.int32)
            bad = jnp.maximum(bad, cnt)
            vm = jnp.where(m >= t, m, NEG)
            im = colbase + mq * L + lane
            res = plsc.sort_key_val(vm, im, descending=True)
            chunk_tops.append((res[0], res[1]))
        fast = _merge_tree(chunk_tops)

        def full_row_top(args, t=t, colbases=colbases, rl=rl):
            del args
            tops = []
            for j in range(TOPK):
                av = jnp.full((L,), NEG, jnp.float32)
                ai = jnp.zeros((L,), jnp.int32)
                for qq in range(CHUNK // L):
                    v = cand_v[rl * TOPK + j, pl.ds(qq * L, L)]
                    vm = jnp.where(v >= t, v, NEG)
                    im = colbases[j] + qq * L + lane
                    sv2, si2 = plsc.sort_key_val(vm, im, descending=True)
                    cv = jnp.where(lane_lt8, av, _hi8(sv2))
                    ci = jnp.where(lane_lt8, ai, _hi8(si2))
                    av, ai = plsc.sort_key_val(cv, ci, descending=True)
                tops.append((av, ai))
            return _merge_tree(tops)

        multi = plsc.all_reduce_population_count(bad >= 2)[0] > 0
        bv, bi = lax.cond(multi, full_row_top, lambda args: args, fast)
        e = jnp.where(lane_lt8, jnp.exp(bv - jnp.max(bv)), 0.0)
        p = e / jnp.sum(e)
        key = jnp.where(lane_lt8, bi, jnp.int32(N))
        sk, sp = plsc.sort_key_val(key, p, descending=False)
        plsc.store_compressed(oc_v.at[pl.ds(r * TOPK, L)], sk, mask=lane_lt8)
        plsc.store_compressed(ov_v.at[pl.ds(r * TOPK, L)], sp, mask=lane_lt8)
        return rbase

    half = BATCH * TOPK // 2
    for b in range(NBATCH):
        rbase = b * BATCH
        lax.fori_loop(0, BATCH, phase_a, rbase)
        c1 = pltpu.async_copy(s_hbm.at[idx_v.at[pl.ds(0, half)]],
                              cand_v.at[pl.ds(0, half)], sem)
        c2 = pltpu.async_copy(s_hbm.at[idx_v.at[pl.ds(half, half)]],
                              cand_v.at[pl.ds(half, half)], sem)
        c1.wait()
        c2.wait()
        lax.fori_loop(0, BATCH, phase_b, rbase)

    pltpu.sync_copy(oc_v.at[pl.ds(0, RW * TOPK)],
                    cols_hbm.at[pl.ds(base * TOPK, RW * TOPK)])
    pltpu.sync_copy(ov_v.at[pl.ds(0, RW * TOPK)],
                    vals_hbm.at[pl.ds(base * TOPK, RW * TOPK)])


_sc_select = functools.partial(
    pl.kernel,
    out_type=[jax.ShapeDtypeStruct((HN * TOPK,), jnp.int32),
              jax.ShapeDtypeStruct((HN * TOPK,), jnp.float32)],
    mesh=plsc.VectorSubcoreMesh(core_axis_name="c", subcore_axis_name="s"),
    compiler_params=pltpu.CompilerParams(needs_layout_passes=False),
    scratch_types=[
        pltpu.VMEM((RW * NCHUNK,), jnp.float32),
        pltpu.VMEM((BATCH * TOPK + L,), jnp.int32),
        pltpu.VMEM((BATCH * L,), jnp.float32),
        pltpu.VMEM((BATCH * TOPK, CHUNK), jnp.float32),
        pltpu.VMEM((RW * TOPK + L,), jnp.int32),
        pltpu.VMEM((RW * TOPK + L,), jnp.float32),
        pltpu.SemaphoreType.DMA,
    ],
)(_sc_select_body)


def _scores_half(q_half, kt):
    return pl.pallas_call(
        _scores_body,
        grid=(HBLK,),
        in_specs=[
            pl.BlockSpec((BLK, KEY_DIM), lambda i: (i, 0)),
            pl.BlockSpec((KEY_DIM, N), lambda i: (0, 0)),
        ],
        out_specs=[
            pl.BlockSpec((NCHUNK, BLK, CHUNK), lambda i: (0, i, 0)),
            pl.BlockSpec((BLK, NCHUNK), lambda i: (i, 0)),
        ],
        out_shape=[
            jax.ShapeDtypeStruct((NCHUNK, HN, CHUNK), jnp.float32),
            jax.ShapeDtypeStruct((HN, NCHUNK), jnp.float32),
        ],
    )(q_half, kt)


@jax.jit
def kernel(U, Wq, bq, Wk, bk):
    q, kt = pl.pallas_call(
        _pool_proj_body,
        grid=(N // BLK,),
        in_specs=[
            pl.BlockSpec((BLK, T, D), lambda i: (i, 0, 0)),
            pl.BlockSpec((KEY_DIM, D), lambda i: (0, 0)),
            pl.BlockSpec((1, KEY_DIM), lambda i: (0, 0)),
            pl.BlockSpec((KEY_DIM, D), lambda i: (0, 0)),
            pl.BlockSpec((1, KEY_DIM), lambda i: (0, 0)),
        ],
        out_specs=[
            pl.BlockSpec((BLK, KEY_DIM), lambda i: (i, 0)),
            pl.BlockSpec((KEY_DIM, BLK), lambda i: (0, i)),
        ],
        out_shape=[
            jax.ShapeDtypeStruct((N, KEY_DIM), jnp.float32),
            jax.ShapeDtypeStruct((KEY_DIM, N), jnp.float32),
        ],
    )(U, Wq, bq.reshape(1, KEY_DIM), Wk, bk.reshape(1, KEY_DIM))

    cols_parts = []
    vals_parts = []
    for h in range(N // HN):
        sh, cmh = _scores_half(q[h * HN:(h + 1) * HN], kt)
        ch, vh = _sc_select(cmh.reshape(-1), sh.reshape(NCHUNK * HN, CHUNK))
        cols_parts.append(ch)
        vals_parts.append(vh)
    cols = jnp.concatenate(cols_parts)
    vals = jnp.concatenate(vals_parts)
    rows = jnp.repeat(jnp.arange(N, dtype=jnp.int32), TOPK)
    indices = jnp.stack([rows.astype(jnp.int64),
                         cols.astype(jnp.int64)], axis=0)
    return indices, vals


# speedup vs baseline: 1.0985x; 1.0383x over previous
import functools
import math

import jax
import jax.numpy as jnp
from jax import lax
from jax.experimental import pallas as pl
from jax.experimental.pallas import tpu as pltpu
from jax.experimental.pallas import tpu_sc as plsc

N, T, D = 4096, 16, 128
KEY_DIM = 64
TOPK = 8
BLK = 256
PBLK = 512
SCALE = 1.0 / math.sqrt(KEY_DIM)

CHUNK = 128
NCHUNK = N // CHUNK
HN = N // 2
HN_LOG = 11
HBLK = HN // BLK
NW = 32
RW = HN // NW
BATCH = 32
NBATCH = RW // BATCH
L = 16
NEG = -3.0e38


def _pool_proj_body(u_ref, wq_ref, bq_ref, wk_ref, bk_ref, q_ref, kt_ref):
    pool = jnp.mean(u_ref[...], axis=1)
    cdims = (((1,), (1,)), ((), ()))
    q_ref[...] = lax.dot_general(pool, wq_ref[...], cdims,
                                 preferred_element_type=jnp.float32) + bq_ref[...]
    k = lax.dot_general(pool, wk_ref[...], cdims,
                        preferred_element_type=jnp.float32) + bk_ref[...]
    kt_ref[...] = k.T


def _scores_body(q_ref, kt_ref, s_ref, cm_ref):
    cms = []
    for c in range(NCHUNK):
        kc = kt_ref[:, c * CHUNK:(c + 1) * CHUNK]
        sc = jnp.dot(q_ref[...], kc,
                     preferred_element_type=jnp.float32) * SCALE
        s_ref[c, :, :] = sc
        cms.append(jnp.max(sc, axis=1, keepdims=True))
    cm_ref[...] = jnp.concatenate(cms, axis=1)


def _hi8(vec):
    return lax.rev(vec, (0,))


def _sc_select_body(cm_hbm, s_hbm, cols_hbm, vals_hbm,
                    cm_v, idx_v, t_v, cand_v, oc_v, ov_v, sem):
    nc = 2
    wid = lax.axis_index("s") * nc + lax.axis_index("c")
    base = wid * RW

    pltpu.sync_copy(cm_hbm.at[pl.ds(base * NCHUNK, RW * NCHUNK)], cm_v)

    lane = lax.iota(jnp.int32, L)
    lane_lt8 = lane < TOPK

    def phase_a(rl, rbase):
        r = rbase + rl
        cm0 = cm_v[pl.ds(r * NCHUNK, L)]
        cm1 = cm_v[pl.ds(r * NCHUNK + L, L)]
        s0, i0 = plsc.sort_key_val(cm0, lane, descending=True)
        s1, i1 = plsc.sort_key_val(cm1, lane + L, descending=True)
        mv = jnp.where(lane_lt8, s0, _hi8(s1))
        mi = jnp.where(lane_lt8, i0, _hi8(i1))
        sv, si = plsc.sort_key_val(mv, mi, descending=True)
        t = jnp.max(jnp.where(lane == TOPK - 1, sv, NEG))
        t_v[pl.ds(rl * L, L)] = jnp.full((L,), t, jnp.float32)
        gidx = si * HN + (base + r)
        plsc.store_compressed(idx_v.at[pl.ds(rl * TOPK, L)], gidx, mask=lane_lt8)
        return rbase

    def _merge_tree(tops):
        while len(tops) > 1:
            nxt = []
            for a, b in zip(tops[0::2], tops[1::2]):
                cv = jnp.where(lane_lt8, a[0], _hi8(b[0]))
                ci = jnp.where(lane_lt8, a[1], _hi8(b[1]))
                res = plsc.sort_key_val(cv, ci, descending=True)
                nxt.append((res[0], res[1]))
            tops = nxt
        return tops[0]

    def phase_b(rl, rbase):
        r = rbase + rl
        t = t_v[pl.ds(rl * L, L)]
        gv = idx_v[pl.ds(rl * TOPK, L)]
        colbases = []
        chunk_tops = []
        bad = jnp.zeros((L,), jnp.int32)
        for j in range(TOPK):
            sid = gv[j]
            colbase = ((sid - (base + r)) >> HN_LOG) * CHUNK
            colbases.append(colbase)
            vs = [cand_v[rl * TOPK + j, pl.ds(qq * L, L)]
                  for qq in range(CHUNK // L)]
            m = vs[0]
            mq = jnp.zeros((L,), jnp.int32)
            cnt = (vs[0] >= t).astype(jnp.int32)
            for qq in range(1, CHUNK // L):
                gt = vs[qq] > m
                m = jnp.where(gt, vs[qq], m)
                mq = jnp.where(gt, qq, mq)
                cnt = cnt + (vs[qq] >= t).astype(jnp.int32)
            bad = jnp.maximum(bad, cnt)
            vm = jnp.where(m >= t, m, NEG)
            im = colbase + mq * L + lane
            res = plsc.sort_key_val(vm, im, descending=True)
            chunk_tops.append((res[0], res[1]))
        fast = _merge_tree(chunk_tops)

        def full_row_top(args, t=t, colbases=colbases, rl=rl):
            del args
            tops = []
            for j in range(TOPK):
                av = jnp.full((L,), NEG, jnp.float32)
                ai = jnp.zeros((L,), jnp.int32)
                for qq in range(CHUNK // L):
                    v = cand_v[rl * TOPK + j, pl.ds(qq * L, L)]
                    vm = jnp.where(v >= t, v, NEG)
                    im = colbases[j] + qq * L + lane
                    sv2, si2 = plsc.sort_key_val(vm, im, descending=True)
                    cv = jnp.where(lane_lt8, av, _hi8(sv2))
                    ci = jnp.where(lane_lt8, ai, _hi8(si2))
                    av, ai = plsc.sort_key_val(cv, ci, descending=True)
                tops.append((av, ai))
            return _merge_tree(tops)

        multi = plsc.all_reduce_population_count(bad >= 2)[0] > 0
        bv, bi = lax.cond(multi, full_row_top, lambda args: args, fast)
        e = jnp.where(lane_lt8, jnp.exp(bv - jnp.max(bv)), 0.0)
        p = e / jnp.sum(e)
        key = jnp.where(lane_lt8, bi, jnp.int32(N))
        sk, sp = plsc.sort_key_val(key, p, descending=False)
        plsc.store_compressed(oc_v.at[pl.ds(r * TOPK, L)], sk, mask=lane_lt8)
        plsc.store_compressed(ov_v.at[pl.ds(r * TOPK, L)], sp, mask=lane_lt8)
        return rbase

    half = BATCH * TOPK // 2
    for b in range(NBATCH):
        rbase = b * BATCH
        lax.fori_loop(0, BATCH, phase_a, rbase)
        c1 = pltpu.async_copy(s_hbm.at[idx_v.at[pl.ds(0, half)]],
                              cand_v.at[pl.ds(0, half)], sem)
        c2 = pltpu.async_copy(s_hbm.at[idx_v.at[pl.ds(half, half)]],
                              cand_v.at[pl.ds(half, half)], sem)
        c1.wait()
        c2.wait()
        lax.fori_loop(0, BATCH, phase_b, rbase)

    pltpu.sync_copy(oc_v.at[pl.ds(0, RW * TOPK)],
                    cols_hbm.at[pl.ds(base * TOPK, RW * TOPK)])
    pltpu.sync_copy(ov_v.at[pl.ds(0, RW * TOPK)],
                    vals_hbm.at[pl.ds(base * TOPK, RW * TOPK)])


_sc_select = functools.partial(
    pl.kernel,
    out_type=[jax.ShapeDtypeStruct((HN * TOPK,), jnp.int32),
              jax.ShapeDtypeStruct((HN * TOPK,), jnp.float32)],
    mesh=plsc.VectorSubcoreMesh(core_axis_name="c", subcore_axis_name="s"),
    compiler_params=pltpu.CompilerParams(needs_layout_passes=False),
    scratch_types=[
        pltpu.VMEM((RW * NCHUNK,), jnp.float32),
        pltpu.VMEM((BATCH * TOPK + L,), jnp.int32),
        pltpu.VMEM((BATCH * L,), jnp.float32),
        pltpu.VMEM((BATCH * TOPK, CHUNK), jnp.float32),
        pltpu.VMEM((RW * TOPK + L,), jnp.int32),
        pltpu.VMEM((RW * TOPK + L,), jnp.float32),
        pltpu.SemaphoreType.DMA,
    ],
)(_sc_select_body)


def _scores_half(q_half, kt):
    return pl.pallas_call(
        _scores_body,
        grid=(HBLK,),
        in_specs=[
            pl.BlockSpec((BLK, KEY_DIM), lambda i: (i, 0)),
            pl.BlockSpec((KEY_DIM, N), lambda i: (0, 0)),
        ],
        out_specs=[
            pl.BlockSpec((NCHUNK, BLK, CHUNK), lambda i: (0, i, 0)),
            pl.BlockSpec((BLK, NCHUNK), lambda i: (i, 0)),
        ],
        out_shape=[
            jax.ShapeDtypeStruct((NCHUNK, HN, CHUNK), jnp.float32),
            jax.ShapeDtypeStruct((HN, NCHUNK), jnp.float32),
        ],
    )(q_half, kt)


@jax.jit
def kernel(U, Wq, bq, Wk, bk):
    q, kt = pl.pallas_call(
        _pool_proj_body,
        grid=(N // PBLK,),
        in_specs=[
            pl.BlockSpec((PBLK, T, D), lambda i: (i, 0, 0)),
            pl.BlockSpec((KEY_DIM, D), lambda i: (0, 0)),
            pl.BlockSpec((1, KEY_DIM), lambda i: (0, 0)),
            pl.BlockSpec((KEY_DIM, D), lambda i: (0, 0)),
            pl.BlockSpec((1, KEY_DIM), lambda i: (0, 0)),
        ],
        out_specs=[
            pl.BlockSpec((PBLK, KEY_DIM), lambda i: (i, 0)),
            pl.BlockSpec((KEY_DIM, PBLK), lambda i: (0, i)),
        ],
        out_shape=[
            jax.ShapeDtypeStruct((N, KEY_DIM), jnp.float32),
            jax.ShapeDtypeStruct((KEY_DIM, N), jnp.float32),
        ],
    )(U, Wq, bq.reshape(1, KEY_DIM), Wk, bk.reshape(1, KEY_DIM))

    cols_parts = []
    vals_parts = []
    for h in range(N // HN):
        sh, cmh = _scores_half(q[h * HN:(h + 1) * HN], kt)
        ch, vh = _sc_select(cmh.reshape(-1), sh.reshape(NCHUNK * HN, CHUNK))
        cols_parts.append(ch)
        vals_parts.append(vh)
    cols = jnp.concatenate(cols_parts)
    vals = jnp.concatenate(vals_parts)
    rows = jnp.repeat(jnp.arange(N, dtype=jnp.int32), TOPK)
    indices = jnp.stack([rows.astype(jnp.int64),
                         cols.astype(jnp.int64)], axis=0)
    return indices, vals


# speedup vs baseline: 1.1142x; 1.0143x over previous
import functools
import math

import jax
import jax.numpy as jnp
from jax import lax
from jax.experimental import pallas as pl
from jax.experimental.pallas import tpu as pltpu
from jax.experimental.pallas import tpu_sc as plsc

N, T, D = 4096, 16, 128
KEY_DIM = 64
TOPK = 8
BLK = 256
PBLK = 1024
SCALE = 1.0 / math.sqrt(KEY_DIM)

CHUNK = 128
NCHUNK = N // CHUNK
HN = N // 2
HN_LOG = 11
HBLK = HN // BLK
NW = 32
RW = HN // NW
BATCH = 32
NBATCH = RW // BATCH
L = 16
NEG = -3.0e38


def _pool_proj_body(u_ref, wq_ref, bq_ref, wk_ref, bk_ref, q_ref, kt_ref):
    pool = jnp.mean(u_ref[...], axis=1)
    cdims = (((1,), (1,)), ((), ()))
    q_ref[...] = lax.dot_general(pool, wq_ref[...], cdims,
                                 preferred_element_type=jnp.float32) + bq_ref[...]
    k = lax.dot_general(pool, wk_ref[...], cdims,
                        preferred_element_type=jnp.float32) + bk_ref[...]
    kt_ref[...] = k.T


def _scores_body(q_ref, kt_ref, s_ref, cm_ref):
    cms = []
    for c in range(NCHUNK):
        kc = kt_ref[:, c * CHUNK:(c + 1) * CHUNK]
        sc = jnp.dot(q_ref[...], kc,
                     preferred_element_type=jnp.float32) * SCALE
        s_ref[c, :, :] = sc
        cms.append(jnp.max(sc, axis=1, keepdims=True))
    cm_ref[...] = jnp.concatenate(cms, axis=1)


def _hi8(vec):
    return lax.rev(vec, (0,))


def _sc_select_body(cm_hbm, s_hbm, cols_hbm, vals_hbm,
                    cm_v, idx_v, t_v, cand_v, oc_v, ov_v, sem):
    nc = 2
    wid = lax.axis_index("s") * nc + lax.axis_index("c")
    base = wid * RW

    pltpu.sync_copy(cm_hbm.at[pl.ds(base * NCHUNK, RW * NCHUNK)], cm_v)

    lane = lax.iota(jnp.int32, L)
    lane_lt8 = lane < TOPK

    def phase_a(rl, rbase):
        r = rbase + rl
        cm0 = cm_v[pl.ds(r * NCHUNK, L)]
        cm1 = cm_v[pl.ds(r * NCHUNK + L, L)]
        s0, i0 = plsc.sort_key_val(cm0, lane, descending=True)
        s1, i1 = plsc.sort_key_val(cm1, lane + L, descending=True)
        mv = jnp.where(lane_lt8, s0, _hi8(s1))
        mi = jnp.where(lane_lt8, i0, _hi8(i1))
        sv, si = plsc.sort_key_val(mv, mi, descending=True)
        t = jnp.max(jnp.where(lane == TOPK - 1, sv, NEG))
        t_v[pl.ds(rl * L, L)] = jnp.full((L,), t, jnp.float32)
        gidx = si * HN + (base + r)
        plsc.store_compressed(idx_v.at[pl.ds(rl * TOPK, L)], gidx, mask=lane_lt8)
        return rbase

    def _merge_tree(tops):
        while len(tops) > 1:
            nxt = []
            for a, b in zip(tops[0::2], tops[1::2]):
                cv = jnp.where(lane_lt8, a[0], _hi8(b[0]))
                ci = jnp.where(lane_lt8, a[1], _hi8(b[1]))
                res = plsc.sort_key_val(cv, ci, descending=True)
                nxt.append((res[0], res[1]))
            tops = nxt
        return tops[0]

    def phase_b(rl, rbase):
        r = rbase + rl
        t = t_v[pl.ds(rl * L, L)]
        gv = idx_v[pl.ds(rl * TOPK, L)]
        colbases = []
        chunk_tops = []
        bad = jnp.zeros((L,), jnp.int32)
        for j in range(TOPK):
            sid = gv[j]
            colbase = ((sid - (base + r)) >> HN_LOG) * CHUNK
            colbases.append(colbase)
            vs = [cand_v[rl * TOPK + j, pl.ds(qq * L, L)]
                  for qq in range(CHUNK // L)]
            m = vs[0]
            mq = jnp.zeros((L,), jnp.int32)
            cnt = (vs[0] >= t).astype(jnp.int32)
            for qq in range(1, CHUNK // L):
                gt = vs[qq] > m
                m = jnp.where(gt, vs[qq], m)
                mq = jnp.where(gt, qq, mq)
                cnt = cnt + (vs[qq] >= t).astype(jnp.int32)
            bad = jnp.maximum(bad, cnt)
            vm = jnp.where(m >= t, m, NEG)
            im = colbase + mq * L + lane
            res = plsc.sort_key_val(vm, im, descending=True)
            chunk_tops.append((res[0], res[1]))
        fast = _merge_tree(chunk_tops)

        def full_row_top(args, t=t, colbases=colbases, rl=rl):
            del args
            tops = []
            for j in range(TOPK):
                av = jnp.full((L,), NEG, jnp.float32)
                ai = jnp.zeros((L,), jnp.int32)
                for qq in range(CHUNK // L):
                    v = cand_v[rl * TOPK + j, pl.ds(qq * L, L)]
                    vm = jnp.where(v >= t, v, NEG)
                    im = colbases[j] + qq * L + lane
                    sv2, si2 = plsc.sort_key_val(vm, im, descending=True)
                    cv = jnp.where(lane_lt8, av, _hi8(sv2))
                    ci = jnp.where(lane_lt8, ai, _hi8(si2))
                    av, ai = plsc.sort_key_val(cv, ci, descending=True)
                tops.append((av, ai))
            return _merge_tree(tops)

        multi = plsc.all_reduce_population_count(bad >= 2)[0] > 0
        bv, bi = lax.cond(multi, full_row_top, lambda args: args, fast)
        e = jnp.where(lane_lt8, jnp.exp(bv - jnp.max(bv)), 0.0)
        p = e / jnp.sum(e)
        key = jnp.where(lane_lt8, bi, jnp.int32(N))
        sk, sp = plsc.sort_key_val(key, p, descending=False)
        plsc.store_compressed(oc_v.at[pl.ds(r * TOPK, L)], sk, mask=lane_lt8)
        plsc.store_compressed(ov_v.at[pl.ds(r * TOPK, L)], sp, mask=lane_lt8)
        return rbase

    half = BATCH * TOPK // 2
    for b in range(NBATCH):
        rbase = b * BATCH
        lax.fori_loop(0, BATCH, phase_a, rbase)
        c1 = pltpu.async_copy(s_hbm.at[idx_v.at[pl.ds(0, half)]],
                              cand_v.at[pl.ds(0, half)], sem)
        c2 = pltpu.async_copy(s_hbm.at[idx_v.at[pl.ds(half, half)]],
                              cand_v.at[pl.ds(half, half)], sem)
        c1.wait()
        c2.wait()
        lax.fori_loop(0, BATCH, phase_b, rbase)

    pltpu.sync_copy(oc_v.at[pl.ds(0, RW * TOPK)],
                    cols_hbm.at[pl.ds(base * TOPK, RW * TOPK)])
    pltpu.sync_copy(ov_v.at[pl.ds(0, RW * TOPK)],
                    vals_hbm.at[pl.ds(base * TOPK, RW * TOPK)])


_sc_select = functools.partial(
    pl.kernel,
    out_type=[jax.ShapeDtypeStruct((HN * TOPK,), jnp.int32),
              jax.ShapeDtypeStruct((HN * TOPK,), jnp.float32)],
    mesh=plsc.VectorSubcoreMesh(core_axis_name="c", subcore_axis_name="s"),
    compiler_params=pltpu.CompilerParams(needs_layout_passes=False),
    scratch_types=[
        pltpu.VMEM((RW * NCHUNK,), jnp.float32),
        pltpu.VMEM((BATCH * TOPK + L,), jnp.int32),
        pltpu.VMEM((BATCH * L,), jnp.float32),
        pltpu.VMEM((BATCH * TOPK, CHUNK), jnp.float32),
        pltpu.VMEM((RW * TOPK + L,), jnp.int32),
        pltpu.VMEM((RW * TOPK + L,), jnp.float32),
        pltpu.SemaphoreType.DMA,
    ],
)(_sc_select_body)


def _scores_half(q_half, kt):
    return pl.pallas_call(
        _scores_body,
        grid=(HBLK,),
        in_specs=[
            pl.BlockSpec((BLK, KEY_DIM), lambda i: (i, 0)),
            pl.BlockSpec((KEY_DIM, N), lambda i: (0, 0)),
        ],
        out_specs=[
            pl.BlockSpec((NCHUNK, BLK, CHUNK), lambda i: (0, i, 0)),
            pl.BlockSpec((BLK, NCHUNK), lambda i: (i, 0)),
        ],
        out_shape=[
            jax.ShapeDtypeStruct((NCHUNK, HN, CHUNK), jnp.float32),
            jax.ShapeDtypeStruct((HN, NCHUNK), jnp.float32),
        ],
    )(q_half, kt)


@jax.jit
def kernel(U, Wq, bq, Wk, bk):
    q, kt = pl.pallas_call(
        _pool_proj_body,
        grid=(N // PBLK,),
        in_specs=[
            pl.BlockSpec((PBLK, T, D), lambda i: (i, 0, 0)),
            pl.BlockSpec((KEY_DIM, D), lambda i: (0, 0)),
            pl.BlockSpec((1, KEY_DIM), lambda i: (0, 0)),
            pl.BlockSpec((KEY_DIM, D), lambda i: (0, 0)),
            pl.BlockSpec((1, KEY_DIM), lambda i: (0, 0)),
        ],
        out_specs=[
            pl.BlockSpec((PBLK, KEY_DIM), lambda i: (i, 0)),
            pl.BlockSpec((KEY_DIM, PBLK), lambda i: (0, i)),
        ],
        out_shape=[
            jax.ShapeDtypeStruct((N, KEY_DIM), jnp.float32),
            jax.ShapeDtypeStruct((KEY_DIM, N), jnp.float32),
        ],
    )(U, Wq, bq.reshape(1, KEY_DIM), Wk, bk.reshape(1, KEY_DIM))

    cols_parts = []
    vals_parts = []
    for h in range(N // HN):
        sh, cmh = _scores_half(q[h * HN:(h + 1) * HN], kt)
        ch, vh = _sc_select(cmh.reshape(-1), sh.reshape(NCHUNK * HN, CHUNK))
        cols_parts.append(ch)
        vals_parts.append(vh)
    cols = jnp.concatenate(cols_parts)
    vals = jnp.concatenate(vals_parts)
    rows = jnp.repeat(jnp.arange(N, dtype=jnp.int32), TOPK)
    indices = jnp.stack([rows.astype(jnp.int64),
                         cols.astype(jnp.int64)], axis=0)
    return indices, vals
